# baseline (speedup 1.0000x reference)
import functools

import jax
import jax.numpy as jnp
from jax import lax
from jax.experimental import pallas as pl
from jax.experimental.pallas import tpu as pltpu
from jax.experimental.pallas import tpu_sc as plsc

P, L, Q, T = 10000, 1000, 3000, 8
PL_, PQ_, QL_ = 80, 27, 3
D = 32
MAX_MODELS, NUM_POLICIES, MAX_QUEUES = 7, 4, 3
ITERS = 8

NC, NS = 2, 16
NW = NC * NS
LANES = 16
CH = 128

QPAD = 3008
B_PT = 81920
QSUM_PER_W = QPAD // NW
QSUM_ROWS = QSUM_PER_W * PQ_
QSUM_CH = 20

ZS = {'traffic': (1385.4058837890625, 859.8118896484375), 'packets': (1.4015231132507324, 0.8932565450668335), 'eq_lambda': (1350.97119140625, 858.316162109375), 'avg_pkts_lambda': (0.9117304086685181, 0.9723503589630127), 'exp_max_factor': (6.663637638092041, 4.715115070343018), 'pkts_lambda_on': (0.9116322994232178, 1.651275396347046), 'avg_t_off': (1.6649284362792969, 2.356407403945923), 'avg_t_on': (1.6649284362792969, 2.356407403945923), 'ar_a': (0.0, 1.0), 'sigma': (0.0, 1.0), 'capacity': (27611.091796875, 20090.62109375), 'queue_size': (30259.10546875, 21410.095703125)}


def _zs(x, name):
    m, s = ZS[name]
    return (x - m) / s


_SC_MESH = dict(core_axis_name="c", subcore_axis_name="s")
_SC_PARAMS = pltpu.CompilerParams(needs_layout_passes=False,
                                  use_tc_tiling_on_sc=False)


def _wid():
    return lax.axis_index("s") * NC + lax.axis_index("c")


def _sc_gather_rows(table, idx3):
    nchunks = idx3.shape[1]
    rows = nchunks * CH
    mesh = plsc.VectorSubcoreMesh(**_SC_MESH)

    @functools.partial(
        pl.kernel,
        out_type=jax.ShapeDtypeStruct((NW, rows, D), jnp.float32),
        mesh=mesh,
        scratch_types=[
            pltpu.VMEM((nchunks, CH), jnp.int32),
            pltpu.VMEM((rows, D), jnp.float32),
            pltpu.SemaphoreType.DMA,
            pltpu.SemaphoreType.DMA,
        ],
        compiler_params=_SC_PARAMS,
    )
    def k(table_h, idx_h, out_h, idx_v, rows_v, sem, sem_o):
        w = _wid()
        pltpu.sync_copy(idx_h.at[w], idx_v)
        cps = [
            pltpu.async_copy(table_h.at[idx_v.at[j]],
                             rows_v.at[pl.ds(j * CH, CH)], sem)
            for j in range(nchunks)
        ]
        outs = []
        for j in range(nchunks):
            cps[j].wait()
            outs.append(pltpu.async_copy(rows_v.at[pl.ds(j * CH, CH)],
                                         out_h.at[w, pl.ds(j * CH, CH)], sem_o))
        for c in outs:
            c.wait()

    return k(table, idx3)


def _sc_gather_rows2(qtab, ltab, qidx3, lidx3):
    nchunks = qidx3.shape[1]
    rows = nchunks * CH
    NB = 8
    mesh = plsc.VectorSubcoreMesh(**_SC_MESH)
    ot = jax.ShapeDtypeStruct((NW, rows, D), jnp.float32)

    @functools.partial(
        pl.kernel,
        out_type=(ot, ot),
        mesh=mesh,
        scratch_types=[
            pltpu.VMEM((nchunks, CH), jnp.int32),
            pltpu.VMEM((nchunks, CH), jnp.int32),
            pltpu.VMEM((NB * CH, D), jnp.float32),
            pltpu.VMEM((NB * CH, D), jnp.float32),
            pltpu.SemaphoreType.DMA,
            pltpu.SemaphoreType.DMA,
        ],
        compiler_params=_SC_PARAMS,
    )
    def k(qtab_h, ltab_h, qidx_h, lidx_h, qout_h, lout_h,
          qidx_v, lidx_v, qbuf, lbuf, sem_g, sem_o):
        w = _wid()
        pltpu.sync_copy(qidx_h.at[w], qidx_v)
        pltpu.sync_copy(lidx_h.at[w], lidx_v)
        qg = [None] * nchunks
        lg = [None] * nchunks
        qo = [None] * nchunks
        lo = [None] * nchunks

        def fire(tab_h, idx_v, buf, lst, j):
            lst[j] = pltpu.async_copy(
                tab_h.at[idx_v.at[j]],
                buf.at[pl.ds((j % NB) * CH, CH)], sem_g)

        for j in range(min(NB, nchunks)):
            fire(qtab_h, qidx_v, qbuf, qg, j)
            fire(ltab_h, lidx_v, lbuf, lg, j)
        for j in range(nchunks):
            qg[j].wait()
            qo[j] = pltpu.async_copy(
                qbuf.at[pl.ds((j % NB) * CH, CH)],
                qout_h.at[w, pl.ds(j * CH, CH)], sem_o)
            lg[j].wait()
            lo[j] = pltpu.async_copy(
                lbuf.at[pl.ds((j % NB) * CH, CH)],
                lout_h.at[w, pl.ds(j * CH, CH)], sem_o)
            if j + NB < nchunks:
                qo[j].wait()
                fire(qtab_h, qidx_v, qbuf, qg, j + NB)
                lo[j].wait()
                fire(ltab_h, lidx_v, lbuf, lg, j + NB)
        for j in range(max(0, nchunks - NB), nchunks):
            qo[j].wait()
            lo[j].wait()

    return k(qtab, ltab, qidx3, lidx3)


def _sc_gather_sum27(pss_flat, idx3):
    mesh = plsc.VectorSubcoreMesh(**_SC_MESH)

    @functools.partial(
        pl.kernel,
        out_type=jax.ShapeDtypeStruct((NW, QSUM_PER_W, D), jnp.float32),
        mesh=mesh,
        scratch_types=[
            pltpu.VMEM((QSUM_CH, CH), jnp.int32),
            pltpu.VMEM((QSUM_CH * CH, D), jnp.float32),
            pltpu.VMEM((QSUM_PER_W, D), jnp.float32),
            pltpu.SemaphoreType.DMA,
        ],
        compiler_params=_SC_PARAMS,
    )
    def k(pss_h, idx_h, out_h, idx_v, rows_v, out_v, sem):
        w = _wid()
        pltpu.sync_copy(idx_h.at[w], idx_v)
        cps = [
            pltpu.async_copy(pss_h.at[idx_v.at[j]],
                             rows_v.at[pl.ds(j * CH, CH)], sem)
            for j in range(QSUM_CH)
        ]
        for c in cps:
            c.wait()

        def qbody(q, _):
            base = q * PQ_
            acc0 = jnp.zeros((LANES,), jnp.float32)
            acc1 = jnp.zeros((LANES,), jnp.float32)
            for j in range(PQ_):
                acc0 = acc0 + rows_v[base + j, pl.ds(0, LANES)]
                acc1 = acc1 + rows_v[base + j, pl.ds(LANES, LANES)]
            out_v[q, pl.ds(0, LANES)] = acc0
            out_v[q, pl.ds(LANES, LANES)] = acc1
            return 0

        lax.fori_loop(0, QSUM_PER_W, qbody, 0)
        pltpu.sync_copy(out_v, out_h.at[w])

    return k(pss_flat, idx3)


def _sc_scalar_gather(table1d, idx2):
    nt = table1d.shape[0]
    npw = idx2.shape[1]
    mesh = plsc.VectorSubcoreMesh(**_SC_MESH)

    @functools.partial(
        pl.kernel,
        out_type=jax.ShapeDtypeStruct((NW, npw), jnp.float32),
        mesh=mesh,
        scratch_types=[
            pltpu.VMEM((nt,), jnp.float32),
            pltpu.VMEM((npw,), jnp.int32),
            pltpu.VMEM((npw,), jnp.float32),
        ],
        compiler_params=_SC_PARAMS,
    )
    def k(tab_h, idx_h, out_h, tab_v, idx_v, out_v):
        w = _wid()
        pltpu.sync_copy(tab_h, tab_v)
        pltpu.sync_copy(idx_h.at[w], idx_v)
        for g in range(npw // LANES):
            iv = idx_v[pl.ds(g * LANES, LANES)]
            out_v[pl.ds(g * LANES, LANES)] = plsc.load_gather(tab_v, [iv])
        pltpu.sync_copy(out_v, out_h.at[w])

    return k(table1d, idx2)


def _sc_load_sum(traffic1d, idx2):
    nt = traffic1d.shape[0]
    mesh = plsc.VectorSubcoreMesh(**_SC_MESH)

    @functools.partial(
        pl.kernel,
        out_type=jax.ShapeDtypeStruct((NW, 2 * LANES), jnp.float32),
        mesh=mesh,
        scratch_types=[
            pltpu.VMEM((nt,), jnp.float32),
            pltpu.VMEM((2 * PL_ * LANES,), jnp.int32),
            pltpu.VMEM((2 * LANES,), jnp.float32),
        ],
        compiler_params=_SC_PARAMS,
    )
    def k(tab_h, idx_h, out_h, tab_v, idx_v, out_v):
        w = _wid()
        pltpu.sync_copy(tab_h, tab_v)
        pltpu.sync_copy(idx_h.at[w], idx_v)
        for g in range(2):
            acc = jnp.zeros((LANES,), jnp.float32)
            for i in range(PL_):
                iv = idx_v[pl.ds((g * PL_ + i) * LANES, LANES)]
                acc = acc + plsc.load_gather(tab_v, [iv])
            out_v[pl.ds(g * LANES, LANES)] = acc
        pltpu.sync_copy(out_v, out_h.at[w])

    return k(traffic1d, idx2)


def _relu(x):
    return jnp.maximum(x, 0.0)


def _embed_body(path_in, sums, cap, pol_oh, queue_in,
                pw1, pb1, pw2, pb2, lw1, lb1, lw2, lb2, qw1, qb1, qw2, qb2,
                ps_o, ls_o, qs_o):
    x = path_in[...]
    h = _relu(x @ pw1[...] + pb1[...])
    ps_o[...] = _relu(h @ pw2[...] + pb2[...])
    load = sums[...] / cap[...]
    li = jnp.concatenate([load, pol_oh[...]], axis=1)
    h = _relu(li @ lw1[...] + lb1[...])
    ls_o[...] = _relu(h @ lw2[...] + lb2[...])
    qi = queue_in[...]
    h = _relu(qi @ qw1[...] + qb1[...])
    qs_o[...] = _relu(h @ qw2[...] + qb2[...])


def _tc_embed(path_in, sums, cap, pol_oh, queue_in, p):
    outs = [
        jax.ShapeDtypeStruct((P, D), jnp.float32),
        jax.ShapeDtypeStruct((L, D), jnp.float32),
        jax.ShapeDtypeStruct((QPAD, D), jnp.float32),
    ]
    return pl.pallas_call(_embed_body, out_shape=outs)(
        path_in, sums, cap, pol_oh, queue_in,
        p['pe_W1'], p['pe_b1'].reshape(1, D), p['pe_W2'], p['pe_b2'].reshape(1, D),
        p['le_W1'], p['le_b1'].reshape(1, D), p['le_W2'], p['le_b2'].reshape(1, D),
        p['qe_W1'], p['qe_b1'].reshape(1, D), p['qe_W2'], p['qe_b2'].reshape(1, D),
    )


def _gru_math(mx, mh, h):
    z = jax.nn.sigmoid(mx[:, 0:D] + mh[:, 0:D])
    r = jax.nn.sigmoid(mx[:, D:2 * D] + mh[:, D:2 * D])
    hh = jnp.tanh(mx[:, 2 * D:3 * D] + r * mh[:, 2 * D:3 * D])
    return z * h + (1.0 - z) * hh


def _path_gru_body(qg, lg, h0, K, R, b0, b1, pss_o, ht_o):
    bp = h0.shape[0]
    qf = qg[...].reshape(bp * T, D)
    lf = lg[...].reshape(bp * T, D)
    xf = jnp.concatenate([qf, lf], axis=1)
    mx_all = (xf @ K[...] + b0[...]).reshape(bp, T, 3 * D)
    h = h0[...]
    pss_o[:, 0, :] = h
    for t in range(T):
        mx = mx_all[:, t, :]
        mh = h @ R[...] + b1[...]
        h = _gru_math(mx, mh, h)
        pss_o[:, t + 1, :] = h
    ht_o[...] = h


def _tc_path_gru(qg3, lg3, h0, K, R, b0, b1, bp=1000):
    ng = P // bp
    outs = [
        jax.ShapeDtypeStruct((P, T + 1, D), jnp.float32),
        jax.ShapeDtypeStruct((P, D), jnp.float32),
    ]
    return pl.pallas_call(
        _path_gru_body,
        grid=(ng,),
        in_specs=[
            pl.BlockSpec((bp, T, D), lambda i: (i, 0, 0)),
            pl.BlockSpec((bp, T, D), lambda i: (i, 0, 0)),
            pl.BlockSpec((bp, D), lambda i: (i, 0)),
            pl.BlockSpec((2 * D, 3 * D), lambda i: (0, 0)),
            pl.BlockSpec((D, 3 * D), lambda i: (0, 0)),
            pl.BlockSpec((1, 3 * D), lambda i: (0, 0)),
            pl.BlockSpec((1, 3 * D), lambda i: (0, 0)),
        ],
        out_specs=[
            pl.BlockSpec((bp, T + 1, D), lambda i: (i, 0, 0)),
            pl.BlockSpec((bp, D), lambda i: (i, 0)),
        ],
        out_shape=outs,
    )(qg3, lg3, h0, K, R, b0, b1)


def _queue_gru_body(xs, hs, K, R, b0, b1, out):
    mx = xs[...] @ K[...] + b0[...]
    mh = hs[...] @ R[...] + b1[...]
    out[...] = _gru_math(mx, mh, hs[...])


def _tc_queue_gru(xs, hs, K, R, b0, b1):
    return pl.pallas_call(
        _queue_gru_body,
        out_shape=jax.ShapeDtypeStruct((QPAD, D), jnp.float32),
    )(xs, hs, K, R, b0, b1)


def _link_gru_body(qg3, hs, K, R, b0, b1, out):
    h = hs[...]
    for j in range(QL_):
        x = qg3[pl.ds(j * 1024, L), :]
        mx = x @ K[...] + b0[...]
        mh = h @ R[...] + b1[...]
        h = _gru_math(mx, mh, h)
    out[...] = h


def _tc_link_gru(qg3_raw, hs, K, R, b0, b1):
    return pl.pallas_call(
        _link_gru_body,
        out_shape=jax.ShapeDtypeStruct((L, D), jnp.float32),
    )(qg3_raw, hs, K, R, b0, b1)


def _readout_body(pss, capg, lenr, tra, pkt, w1, b1, w2, b2, w3, b3, out):
    bp = out.shape[0]
    qd = jnp.zeros((bp, 1), jnp.float32)
    ts = jnp.zeros((bp, 1), jnp.float32)
    lenv = lenr[...]
    for t in range(T):
        x = pss[:, t + 1, :]
        h = _relu(x @ w1[...] + b1[...])
        h = _relu(h @ w2[...] + b2[...])
        occ = h @ w3[...] + b3[...]
        m = (lenv > t).astype(jnp.float32)
        c = capg[:, pl.ds(t, 1)]
        qd = qd + m * occ / c
        ts = ts + m / c
    out[...] = qd + (tra[...] / pkt[...]) * ts


def _tc_readout(pss, capg, length2, traffic, packets, p, bp=2000):
    ng = P // bp
    return pl.pallas_call(
        _readout_body,
        grid=(ng,),
        in_specs=[
            pl.BlockSpec((bp, T + 1, D), lambda i: (i, 0, 0)),
            pl.BlockSpec((bp, T), lambda i: (i, 0)),
            pl.BlockSpec((bp, 1), lambda i: (i, 0)),
            pl.BlockSpec((bp, 1), lambda i: (i, 0)),
            pl.BlockSpec((bp, 1), lambda i: (i, 0)),
            pl.BlockSpec((D, 16), lambda i: (0, 0)),
            pl.BlockSpec((1, 16), lambda i: (0, 0)),
            pl.BlockSpec((16, 16), lambda i: (0, 0)),
            pl.BlockSpec((1, 16), lambda i: (0, 0)),
            pl.BlockSpec((16, 1), lambda i: (0, 0)),
            pl.BlockSpec((1, 1), lambda i: (0, 0)),
        ],
        out_specs=pl.BlockSpec((bp, 1), lambda i: (i, 0)),
        out_shape=jax.ShapeDtypeStruct((P, 1), jnp.float32),
    )(pss, capg, length2, traffic, packets,
      p['ro_W1'], p['ro_b1'].reshape(1, 16), p['ro_W2'], p['ro_b2'].reshape(1, 16),
      p['ro_W3'], p['ro_b3'].reshape(1, 1))


def _pad_to(x, n):
    return jnp.pad(x, ((0, n - x.shape[0]),))


def _chunk_idx(flat, total, nchunks):
    return _pad_to(flat, total).reshape(NW, nchunks, CH)


def kernel(traffic, packets, eq_lambda, avg_pkts_lambda, exp_max_factor,
           pkts_lambda_on, avg_t_off, avg_t_on, ar_a, sigma, capacity,
           queue_size, weight, length, model, policy, priority,
           queue_to_path, link_to_path, path_to_link, path_to_queue,
           queue_to_link, params):
    p = params

    qtp_idx3 = _chunk_idx(queue_to_path.reshape(-1), B_PT, B_PT // (NW * CH))
    ltp_idx3 = _chunk_idx(link_to_path.reshape(-1), B_PT, B_PT // (NW * CH))

    ptq_flat = (path_to_queue[:, :, 0] * (T + 1) + path_to_queue[:, :, 1])
    ptq_flat = jnp.pad(ptq_flat, ((0, QPAD - Q), (0, 0))).reshape(NW, QSUM_ROWS)
    ptq_idx3 = jnp.pad(ptq_flat, ((0, 0), (0, QSUM_CH * CH - QSUM_ROWS))
                       ).reshape(NW, QSUM_CH, CH)

    qtl_t = jnp.pad(queue_to_link.T, ((0, 0), (0, 1024 - L))).reshape(-1)
    qtl_idx3 = _pad_to(qtl_t, NW * CH).reshape(NW, 1, CH)

    capg_idx2 = _pad_to(link_to_path.reshape(-1), B_PT).reshape(NW, B_PT // NW)

    ptl0 = jnp.pad(path_to_link[:, :, 0], ((0, 1024 - L), (0, 0)))
    load_idx2 = ptl0.reshape(NW, 2, LANES, PL_).transpose(0, 1, 3, 2
                                                          ).reshape(NW, -1)

    model_oh = jax.nn.one_hot(model, MAX_MODELS, dtype=jnp.float32)
    policy_oh = jax.nn.one_hot(policy, NUM_POLICIES, dtype=jnp.float32)
    priority_oh = jax.nn.one_hot(priority, MAX_QUEUES, dtype=jnp.float32)
    path_in = jnp.concatenate([
        _zs(traffic, 'traffic'), _zs(packets, 'packets'), model_oh,
        _zs(eq_lambda, 'eq_lambda'), _zs(avg_pkts_lambda, 'avg_pkts_lambda'),
        _zs(exp_max_factor, 'exp_max_factor'), _zs(pkts_lambda_on, 'pkts_lambda_on'),
        _zs(avg_t_off, 'avg_t_off'), _zs(avg_t_on, 'avg_t_on'),
        _zs(ar_a, 'ar_a'), _zs(sigma, 'sigma')], axis=1)
    queue_in = jnp.concatenate([
        _zs(queue_size, 'queue_size'), priority_oh, weight], axis=1)
    queue_in = jnp.pad(queue_in, ((0, QPAD - Q), (0, 0)))

    sums_raw = _sc_load_sum(traffic.reshape(-1), load_idx2)
    sums = sums_raw.reshape(-1)[:L].reshape(L, 1)
    capg_raw = _sc_scalar_gather(capacity.reshape(-1), capg_idx2)
    capg = capg_raw.reshape(-1)[:P * T].reshape(P, T)

    path_state, link_state, queue_state = _tc_embed(
        path_in, sums, capacity, policy_oh, queue_in, p)

    pb0 = p['path_b'][0:1, :]
    pb1 = p['path_b'][1:2, :]
    qb0 = p['queue_b'][0:1, :]
    qb1 = p['queue_b'][1:2, :]
    lb0 = p['link_b'][0:1, :]
    lb1 = p['link_b'][1:2, :]

    pss = None
    for _ in range(ITERS):
        qg_raw, lg_raw = _sc_gather_rows2(
            queue_state, link_state, qtp_idx3, ltp_idx3)
        qg3 = qg_raw.reshape(B_PT, D)[: P * T].reshape(P, T, D)
        lg3 = lg_raw.reshape(B_PT, D)[: P * T].reshape(P, T, D)
        pss, path_state = _tc_path_gru(
            qg3, lg3, path_state, p['path_K'], p['path_R'], pb0, pb1)
        psum_raw = _sc_gather_sum27(pss.reshape(P * (T + 1), D), ptq_idx3)
        queue_state = _tc_queue_gru(
            psum_raw.reshape(QPAD, D), queue_state,
            p['queue_K'], p['queue_R'], qb0, qb1)
        qg3l_raw = _sc_gather_rows(queue_state, qtl_idx3)
        link_state = _tc_link_gru(
            qg3l_raw.reshape(NW * CH, D), link_state,
            p['link_K'], p['link_R'], lb0, lb1)

    return _tc_readout(pss, capg, length.reshape(P, 1).astype(jnp.int32),
                       traffic, packets, p)


# speedup vs baseline: 1.5283x; 1.5283x over previous
import functools

import jax
import jax.numpy as jnp
from jax import lax
from jax.experimental import pallas as pl
from jax.experimental.pallas import tpu as pltpu
from jax.experimental.pallas import tpu_sc as plsc

P, L, Q, T = 10000, 1000, 3000, 8
PL_, PQ_, QL_ = 80, 27, 3
D = 32
MAX_MODELS, NUM_POLICIES, MAX_QUEUES = 7, 4, 3
ITERS = 8

NC, NS = 2, 16
NW = NC * NS
LANES = 16
CH = 128

QPAD = 3008
B_PT = 81920
QSUM_PER_W = QPAD // NW
QSUM_ROWS = QSUM_PER_W * PQ_
QSUM_CH = 20

ZS = {'traffic': (1385.4058837890625, 859.8118896484375), 'packets': (1.4015231132507324, 0.8932565450668335), 'eq_lambda': (1350.97119140625, 858.316162109375), 'avg_pkts_lambda': (0.9117304086685181, 0.9723503589630127), 'exp_max_factor': (6.663637638092041, 4.715115070343018), 'pkts_lambda_on': (0.9116322994232178, 1.651275396347046), 'avg_t_off': (1.6649284362792969, 2.356407403945923), 'avg_t_on': (1.6649284362792969, 2.356407403945923), 'ar_a': (0.0, 1.0), 'sigma': (0.0, 1.0), 'capacity': (27611.091796875, 20090.62109375), 'queue_size': (30259.10546875, 21410.095703125)}


def _zs(x, name):
    m, s = ZS[name]
    return (x - m) / s


_SC_MESH = dict(core_axis_name="c", subcore_axis_name="s")
_SC_PARAMS = pltpu.CompilerParams(needs_layout_passes=False,
                                  use_tc_tiling_on_sc=False)


def _wid():
    return lax.axis_index("s") * NC + lax.axis_index("c")


def _sc_gather_rows(table, idx3):
    nchunks = idx3.shape[1]
    rows = nchunks * CH
    mesh = plsc.VectorSubcoreMesh(**_SC_MESH)

    @functools.partial(
        pl.kernel,
        out_type=jax.ShapeDtypeStruct((NW, rows, D), jnp.float32),
        mesh=mesh,
        scratch_types=[
            pltpu.VMEM((nchunks, CH), jnp.int32),
            pltpu.VMEM((rows, D), jnp.float32),
            pltpu.SemaphoreType.DMA,
            pltpu.SemaphoreType.DMA,
        ],
        compiler_params=_SC_PARAMS,
    )
    def k(table_h, idx_h, out_h, idx_v, rows_v, sem, sem_o):
        w = _wid()
        pltpu.sync_copy(idx_h.at[w], idx_v)
        cps = [
            pltpu.async_copy(table_h.at[idx_v.at[j]],
                             rows_v.at[pl.ds(j * CH, CH)], sem)
            for j in range(nchunks)
        ]
        outs = []
        for j in range(nchunks):
            cps[j].wait()
            outs.append(pltpu.async_copy(rows_v.at[pl.ds(j * CH, CH)],
                                         out_h.at[w, pl.ds(j * CH, CH)], sem_o))
        for c in outs:
            c.wait()

    return k(table, idx3)


def _sc_gather_rows2(qtab, ltab, qidx3, lidx3):
    nchunks = qidx3.shape[1]
    rows = nchunks * CH
    NB = 8
    mesh = plsc.VectorSubcoreMesh(**_SC_MESH)
    ot = jax.ShapeDtypeStruct((NW, rows, D), jnp.float32)

    @functools.partial(
        pl.kernel,
        out_type=(ot, ot),
        mesh=mesh,
        scratch_types=[
            pltpu.VMEM((nchunks, CH), jnp.int32),
            pltpu.VMEM((nchunks, CH), jnp.int32),
            pltpu.VMEM((NB * CH, D), jnp.float32),
            pltpu.VMEM((NB * CH, D), jnp.float32),
            pltpu.SemaphoreType.DMA,
            pltpu.SemaphoreType.DMA,
        ],
        compiler_params=_SC_PARAMS,
    )
    def k(qtab_h, ltab_h, qidx_h, lidx_h, qout_h, lout_h,
          qidx_v, lidx_v, qbuf, lbuf, sem_g, sem_o):
        w = _wid()
        pltpu.sync_copy(qidx_h.at[w], qidx_v)
        pltpu.sync_copy(lidx_h.at[w], lidx_v)
        qg = [None] * nchunks
        lg = [None] * nchunks
        qo = [None] * nchunks
        lo = [None] * nchunks

        def fire(tab_h, idx_v, buf, lst, j):
            lst[j] = pltpu.async_copy(
                tab_h.at[idx_v.at[j]],
                buf.at[pl.ds((j % NB) * CH, CH)], sem_g)

        for j in range(min(NB, nchunks)):
            fire(qtab_h, qidx_v, qbuf, qg, j)
            fire(ltab_h, lidx_v, lbuf, lg, j)
        for j in range(nchunks):
            qg[j].wait()
            qo[j] = pltpu.async_copy(
                qbuf.at[pl.ds((j % NB) * CH, CH)],
                qout_h.at[w, pl.ds(j * CH, CH)], sem_o)
            lg[j].wait()
            lo[j] = pltpu.async_copy(
                lbuf.at[pl.ds((j % NB) * CH, CH)],
                lout_h.at[w, pl.ds(j * CH, CH)], sem_o)
            if j + NB < nchunks:
                qo[j].wait()
                fire(qtab_h, qidx_v, qbuf, qg, j + NB)
                lo[j].wait()
                fire(ltab_h, lidx_v, lbuf, lg, j + NB)
        for j in range(max(0, nchunks - NB), nchunks):
            qo[j].wait()
            lo[j].wait()

    return k(qtab, ltab, qidx3, lidx3)


def _sc_gather_sum27(pss_flat, idx3):
    mesh = plsc.VectorSubcoreMesh(**_SC_MESH)

    @functools.partial(
        pl.kernel,
        out_type=jax.ShapeDtypeStruct((NW, QSUM_PER_W, D), jnp.float32),
        mesh=mesh,
        scratch_types=[
            pltpu.VMEM((QSUM_CH, CH), jnp.int32),
            pltpu.VMEM((QSUM_CH * CH, D), jnp.float32),
            pltpu.VMEM((QSUM_PER_W, D), jnp.float32),
            pltpu.SemaphoreType.DMA,
        ],
        compiler_params=_SC_PARAMS,
    )
    def k(pss_h, idx_h, out_h, idx_v, rows_v, out_v, sem):
        w = _wid()
        pltpu.sync_copy(idx_h.at[w], idx_v)
        cps = [
            pltpu.async_copy(pss_h.at[idx_v.at[j]],
                             rows_v.at[pl.ds(j * CH, CH)], sem)
            for j in range(QSUM_CH)
        ]
        for c in cps:
            c.wait()

        def qbody(q, _):
            base = q * PQ_
            acc0 = jnp.zeros((LANES,), jnp.float32)
            acc1 = jnp.zeros((LANES,), jnp.float32)
            for j in range(PQ_):
                acc0 = acc0 + rows_v[base + j, pl.ds(0, LANES)]
                acc1 = acc1 + rows_v[base + j, pl.ds(LANES, LANES)]
            out_v[q, pl.ds(0, LANES)] = acc0
            out_v[q, pl.ds(LANES, LANES)] = acc1
            return 0

        lax.fori_loop(0, QSUM_PER_W, qbody, 0)
        pltpu.sync_copy(out_v, out_h.at[w])

    return k(pss_flat, idx3)


def _sc_scalar_gather(table1d, idx2):
    nt = table1d.shape[0]
    npw = idx2.shape[1]
    mesh = plsc.VectorSubcoreMesh(**_SC_MESH)

    @functools.partial(
        pl.kernel,
        out_type=jax.ShapeDtypeStruct((NW, npw), jnp.float32),
        mesh=mesh,
        scratch_types=[
            pltpu.VMEM((nt,), jnp.float32),
            pltpu.VMEM((npw,), jnp.int32),
            pltpu.VMEM((npw,), jnp.float32),
        ],
        compiler_params=_SC_PARAMS,
    )
    def k(tab_h, idx_h, out_h, tab_v, idx_v, out_v):
        w = _wid()
        pltpu.sync_copy(tab_h, tab_v)
        pltpu.sync_copy(idx_h.at[w], idx_v)
        for g in range(npw // LANES):
            iv = idx_v[pl.ds(g * LANES, LANES)]
            out_v[pl.ds(g * LANES, LANES)] = plsc.load_gather(tab_v, [iv])
        pltpu.sync_copy(out_v, out_h.at[w])

    return k(table1d, idx2)


def _sc_load_sum(traffic1d, idx2):
    nt = traffic1d.shape[0]
    mesh = plsc.VectorSubcoreMesh(**_SC_MESH)

    @functools.partial(
        pl.kernel,
        out_type=jax.ShapeDtypeStruct((NW, 2 * LANES), jnp.float32),
        mesh=mesh,
        scratch_types=[
            pltpu.VMEM((nt,), jnp.float32),
            pltpu.VMEM((2 * PL_ * LANES,), jnp.int32),
            pltpu.VMEM((2 * LANES,), jnp.float32),
        ],
        compiler_params=_SC_PARAMS,
    )
    def k(tab_h, idx_h, out_h, tab_v, idx_v, out_v):
        w = _wid()
        pltpu.sync_copy(tab_h, tab_v)
        pltpu.sync_copy(idx_h.at[w], idx_v)
        for g in range(2):
            acc = jnp.zeros((LANES,), jnp.float32)
            for i in range(PL_):
                iv = idx_v[pl.ds((g * PL_ + i) * LANES, LANES)]
                acc = acc + plsc.load_gather(tab_v, [iv])
            out_v[pl.ds(g * LANES, LANES)] = acc
        pltpu.sync_copy(out_v, out_h.at[w])

    return k(traffic1d, idx2)


def _relu(x):
    return jnp.maximum(x, 0.0)


def _embed_body(path_in, sums, cap, pol_oh, queue_in,
                pw1, pb1, pw2, pb2, lw1, lb1, lw2, lb2, qw1, qb1, qw2, qb2,
                ps_o, ls_o, qs_o):
    x = path_in[...]
    h = _relu(x @ pw1[...] + pb1[...])
    ps_o[...] = _relu(h @ pw2[...] + pb2[...])
    load = sums[...] / cap[...]
    li = jnp.concatenate([load, pol_oh[...]], axis=1)
    h = _relu(li @ lw1[...] + lb1[...])
    ls_o[...] = _relu(h @ lw2[...] + lb2[...])
    qi = queue_in[...]
    h = _relu(qi @ qw1[...] + qb1[...])
    qs_o[...] = _relu(h @ qw2[...] + qb2[...])


def _tc_embed(path_in, sums, cap, pol_oh, queue_in, p):
    outs = [
        jax.ShapeDtypeStruct((P, D), jnp.float32),
        jax.ShapeDtypeStruct((L, D), jnp.float32),
        jax.ShapeDtypeStruct((QPAD, D), jnp.float32),
    ]
    return pl.pallas_call(_embed_body, out_shape=outs)(
        path_in, sums, cap, pol_oh, queue_in,
        p['pe_W1'], p['pe_b1'].reshape(1, D), p['pe_W2'], p['pe_b2'].reshape(1, D),
        p['le_W1'], p['le_b1'].reshape(1, D), p['le_W2'], p['le_b2'].reshape(1, D),
        p['qe_W1'], p['qe_b1'].reshape(1, D), p['qe_W2'], p['qe_b2'].reshape(1, D),
    )


def _gru_math(mx, mh, h):
    z = jax.nn.sigmoid(mx[:, 0:D] + mh[:, 0:D])
    r = jax.nn.sigmoid(mx[:, D:2 * D] + mh[:, D:2 * D])
    hh = jnp.tanh(mx[:, 2 * D:3 * D] + r * mh[:, 2 * D:3 * D])
    return z * h + (1.0 - z) * hh


def _path_gru_body(qg, lg, h0, K, R, b0, b1, pss_o, ht_o):
    h = h0[...]
    pss_o[:, 0, :] = h
    for t in range(T):
        x = jnp.concatenate([qg[:, t, :], lg[:, t, :]], axis=1)
        mx = x @ K[...] + b0[...]
        mh = h @ R[...] + b1[...]
        h = _gru_math(mx, mh, h)
        pss_o[:, t + 1, :] = h
    ht_o[...] = h


def _tc_path_gru(qg3, lg3, h0, K, R, b0, b1, bp=1000):
    ng = P // bp
    outs = [
        jax.ShapeDtypeStruct((P, T + 1, D), jnp.float32),
        jax.ShapeDtypeStruct((P, D), jnp.float32),
    ]
    return pl.pallas_call(
        _path_gru_body,
        grid=(ng,),
        in_specs=[
            pl.BlockSpec((bp, T, D), lambda i: (i, 0, 0)),
            pl.BlockSpec((bp, T, D), lambda i: (i, 0, 0)),
            pl.BlockSpec((bp, D), lambda i: (i, 0)),
            pl.BlockSpec((2 * D, 3 * D), lambda i: (0, 0)),
            pl.BlockSpec((D, 3 * D), lambda i: (0, 0)),
            pl.BlockSpec((1, 3 * D), lambda i: (0, 0)),
            pl.BlockSpec((1, 3 * D), lambda i: (0, 0)),
        ],
        out_specs=[
            pl.BlockSpec((bp, T + 1, D), lambda i: (i, 0, 0)),
            pl.BlockSpec((bp, D), lambda i: (i, 0)),
        ],
        out_shape=outs,
    )(qg3, lg3, h0, K, R, b0, b1)


def _queue_gru_body(xs, hs, K, R, b0, b1, out):
    mx = xs[...] @ K[...] + b0[...]
    mh = hs[...] @ R[...] + b1[...]
    out[...] = _gru_math(mx, mh, hs[...])


def _tc_queue_gru(xs, hs, K, R, b0, b1):
    return pl.pallas_call(
        _queue_gru_body,
        out_shape=jax.ShapeDtypeStruct((QPAD, D), jnp.float32),
    )(xs, hs, K, R, b0, b1)


def _link_gru_body(qg3, hs, K, R, b0, b1, out):
    h = hs[...]
    for j in range(QL_):
        x = qg3[pl.ds(j * 1024, L), :]
        mx = x @ K[...] + b0[...]
        mh = h @ R[...] + b1[...]
        h = _gru_math(mx, mh, h)
    out[...] = h


def _tc_link_gru(qg3_raw, hs, K, R, b0, b1):
    return pl.pallas_call(
        _link_gru_body,
        out_shape=jax.ShapeDtypeStruct((L, D), jnp.float32),
    )(qg3_raw, hs, K, R, b0, b1)


def _readout_body(pss, capg, lenr, tra, pkt, w1, b1, w2, b2, w3, b3, out):
    bp = out.shape[0]
    qd = jnp.zeros((bp, 1), jnp.float32)
    ts = jnp.zeros((bp, 1), jnp.float32)
    lenv = lenr[...]
    for t in range(T):
        x = pss[:, t + 1, :]
        h = _relu(x @ w1[...] + b1[...])
        h = _relu(h @ w2[...] + b2[...])
        occ = h @ w3[...] + b3[...]
        m = (lenv > t).astype(jnp.float32)
        c = capg[:, pl.ds(t, 1)]
        qd = qd + m * occ / c
        ts = ts + m / c
    out[...] = qd + (tra[...] / pkt[...]) * ts


def _tc_readout(pss, capg, length2, traffic, packets, p, bp=2000):
    ng = P // bp
    return pl.pallas_call(
        _readout_body,
        grid=(ng,),
        in_specs=[
            pl.BlockSpec((bp, T + 1, D), lambda i: (i, 0, 0)),
            pl.BlockSpec((bp, T), lambda i: (i, 0)),
            pl.BlockSpec((bp, 1), lambda i: (i, 0)),
            pl.BlockSpec((bp, 1), lambda i: (i, 0)),
            pl.BlockSpec((bp, 1), lambda i: (i, 0)),
            pl.BlockSpec((D, 16), lambda i: (0, 0)),
            pl.BlockSpec((1, 16), lambda i: (0, 0)),
            pl.BlockSpec((16, 16), lambda i: (0, 0)),
            pl.BlockSpec((1, 16), lambda i: (0, 0)),
            pl.BlockSpec((16, 1), lambda i: (0, 0)),
            pl.BlockSpec((1, 1), lambda i: (0, 0)),
        ],
        out_specs=pl.BlockSpec((bp, 1), lambda i: (i, 0)),
        out_shape=jax.ShapeDtypeStruct((P, 1), jnp.float32),
    )(pss, capg, length2, traffic, packets,
      p['ro_W1'], p['ro_b1'].reshape(1, 16), p['ro_W2'], p['ro_b2'].reshape(1, 16),
      p['ro_W3'], p['ro_b3'].reshape(1, 1))


def _pad_to(x, n):
    return jnp.pad(x, ((0, n - x.shape[0]),))


def _chunk_idx(flat, total, nchunks):
    return _pad_to(flat, total).reshape(NW, nchunks, CH)


def kernel(traffic, packets, eq_lambda, avg_pkts_lambda, exp_max_factor,
           pkts_lambda_on, avg_t_off, avg_t_on, ar_a, sigma, capacity,
           queue_size, weight, length, model, policy, priority,
           queue_to_path, link_to_path, path_to_link, path_to_queue,
           queue_to_link, params):
    p = params

    qtp_idx3 = _chunk_idx(queue_to_path.reshape(-1), B_PT, B_PT // (NW * CH))
    ltp_idx3 = _chunk_idx(link_to_path.reshape(-1), B_PT, B_PT // (NW * CH))

    ptq_flat = (path_to_queue[:, :, 0] * (T + 1) + path_to_queue[:, :, 1])
    ptq_flat = jnp.pad(ptq_flat, ((0, QPAD - Q), (0, 0))).reshape(NW, QSUM_ROWS)
    ptq_idx3 = jnp.pad(ptq_flat, ((0, 0), (0, QSUM_CH * CH - QSUM_ROWS))
                       ).reshape(NW, QSUM_CH, CH)

    qtl_t = jnp.pad(queue_to_link.T, ((0, 0), (0, 1024 - L))).reshape(-1)
    qtl_idx3 = _pad_to(qtl_t, NW * CH).reshape(NW, 1, CH)

    capg_idx2 = _pad_to(link_to_path.reshape(-1), B_PT).reshape(NW, B_PT // NW)

    ptl0 = jnp.pad(path_to_link[:, :, 0], ((0, 1024 - L), (0, 0)))
    load_idx2 = ptl0.reshape(NW, 2, LANES, PL_).transpose(0, 1, 3, 2
                                                          ).reshape(NW, -1)

    model_oh = jax.nn.one_hot(model, MAX_MODELS, dtype=jnp.float32)
    policy_oh = jax.nn.one_hot(policy, NUM_POLICIES, dtype=jnp.float32)
    priority_oh = jax.nn.one_hot(priority, MAX_QUEUES, dtype=jnp.float32)
    path_in = jnp.concatenate([
        _zs(traffic, 'traffic'), _zs(packets, 'packets'), model_oh,
        _zs(eq_lambda, 'eq_lambda'), _zs(avg_pkts_lambda, 'avg_pkts_lambda'),
        _zs(exp_max_factor, 'exp_max_factor'), _zs(pkts_lambda_on, 'pkts_lambda_on'),
        _zs(avg_t_off, 'avg_t_off'), _zs(avg_t_on, 'avg_t_on'),
        _zs(ar_a, 'ar_a'), _zs(sigma, 'sigma')], axis=1)
    queue_in = jnp.concatenate([
        _zs(queue_size, 'queue_size'), priority_oh, weight], axis=1)
    queue_in = jnp.pad(queue_in, ((0, QPAD - Q), (0, 0)))

    sums_raw = _sc_load_sum(traffic.reshape(-1), load_idx2)
    sums = sums_raw.reshape(-1)[:L].reshape(L, 1)
    capg_raw = _sc_scalar_gather(capacity.reshape(-1), capg_idx2)
    capg = capg_raw.reshape(-1)[:P * T].reshape(P, T)

    path_state, link_state, queue_state = _tc_embed(
        path_in, sums, capacity, policy_oh, queue_in, p)

    pb0 = p['path_b'][0:1, :]
    pb1 = p['path_b'][1:2, :]
    qb0 = p['queue_b'][0:1, :]
    qb1 = p['queue_b'][1:2, :]
    lb0 = p['link_b'][0:1, :]
    lb1 = p['link_b'][1:2, :]

    pss = None
    for _ in range(ITERS):
        qg_raw, lg_raw = _sc_gather_rows2(
            queue_state, link_state, qtp_idx3, ltp_idx3)
        qg3 = qg_raw.reshape(B_PT, D)[: P * T].reshape(P, T, D)
        lg3 = lg_raw.reshape(B_PT, D)[: P * T].reshape(P, T, D)
        pss, path_state = _tc_path_gru(
            qg3, lg3, path_state, p['path_K'], p['path_R'], pb0, pb1)
        psum_raw = _sc_gather_sum27(pss.reshape(P * (T + 1), D), ptq_idx3)
        queue_state = _tc_queue_gru(
            psum_raw.reshape(QPAD, D), queue_state,
            p['queue_K'], p['queue_R'], qb0, qb1)
        qg3l_raw = _sc_gather_rows(queue_state, qtl_idx3)
        link_state = _tc_link_gru(
            qg3l_raw.reshape(NW * CH, D), link_state,
            p['link_K'], p['link_R'], lb0, lb1)

    return _tc_readout(pss, capg, length.reshape(P, 1).astype(jnp.int32),
                       traffic, packets, p)


# speedup vs baseline: 2.3291x; 1.5240x over previous
import functools

import jax
import jax.numpy as jnp
from jax import lax
from jax.experimental import pallas as pl
from jax.experimental.pallas import tpu as pltpu
from jax.experimental.pallas import tpu_sc as plsc

P, L, Q, T = 10000, 1000, 3000, 8
PL_, PQ_, QL_ = 80, 27, 3
D = 32
MAX_MODELS, NUM_POLICIES, MAX_QUEUES = 7, 4, 3
ITERS = 8

NC, NS = 2, 16
NW = NC * NS
LANES = 16
CH = 128

QPAD = 3008
PT_PAD = 10240
B_PT = 81920
QSUM_PER_W = QPAD // NW
QSUM_ROWS = QSUM_PER_W * PQ_
QSUM_CH = 20

ZS = {'traffic': (1385.4058837890625, 859.8118896484375), 'packets': (1.4015231132507324, 0.8932565450668335), 'eq_lambda': (1350.97119140625, 858.316162109375), 'avg_pkts_lambda': (0.9117304086685181, 0.9723503589630127), 'exp_max_factor': (6.663637638092041, 4.715115070343018), 'pkts_lambda_on': (0.9116322994232178, 1.651275396347046), 'avg_t_off': (1.6649284362792969, 2.356407403945923), 'avg_t_on': (1.6649284362792969, 2.356407403945923), 'ar_a': (0.0, 1.0), 'sigma': (0.0, 1.0), 'capacity': (27611.091796875, 20090.62109375), 'queue_size': (30259.10546875, 21410.095703125)}


def _zs(x, name):
    m, s = ZS[name]
    return (x - m) / s


_SC_MESH = dict(core_axis_name="c", subcore_axis_name="s")
_SC_PARAMS = pltpu.CompilerParams(needs_layout_passes=False,
                                  use_tc_tiling_on_sc=False)


def _wid():
    return lax.axis_index("s") * NC + lax.axis_index("c")


def _sc_gather_rows(table, idx3):
    nchunks = idx3.shape[1]
    rows = nchunks * CH
    mesh = plsc.VectorSubcoreMesh(**_SC_MESH)

    @functools.partial(
        pl.kernel,
        out_type=jax.ShapeDtypeStruct((NW, rows, D), jnp.float32),
        mesh=mesh,
        scratch_types=[
            pltpu.VMEM((nchunks, CH), jnp.int32),
            pltpu.VMEM((rows, D), jnp.float32),
            pltpu.SemaphoreType.DMA,
            pltpu.SemaphoreType.DMA,
        ],
        compiler_params=_SC_PARAMS,
    )
    def k(table_h, idx_h, out_h, idx_v, rows_v, sem, sem_o):
        w = _wid()
        pltpu.sync_copy(idx_h.at[w], idx_v)
        cps = [
            pltpu.async_copy(table_h.at[idx_v.at[j]],
                             rows_v.at[pl.ds(j * CH, CH)], sem)
            for j in range(nchunks)
        ]
        outs = []
        for j in range(nchunks):
            cps[j].wait()
            outs.append(pltpu.async_copy(rows_v.at[pl.ds(j * CH, CH)],
                                         out_h.at[w, pl.ds(j * CH, CH)], sem_o))
        for c in outs:
            c.wait()

    return k(table, idx3)


def _sc_gather_rows2(qtab, ltab, qidx3, lidx3):
    nchunks = qidx3.shape[1]
    rows = nchunks * CH
    NB = 8
    mesh = plsc.VectorSubcoreMesh(**_SC_MESH)
    ot = jax.ShapeDtypeStruct((NW, rows, D), jnp.float32)

    @functools.partial(
        pl.kernel,
        out_type=(ot, ot),
        mesh=mesh,
        scratch_types=[
            pltpu.VMEM((nchunks, CH), jnp.int32),
            pltpu.VMEM((nchunks, CH), jnp.int32),
            pltpu.VMEM((NB * CH, D), jnp.float32),
            pltpu.VMEM((NB * CH, D), jnp.float32),
            pltpu.SemaphoreType.DMA,
            pltpu.SemaphoreType.DMA,
        ],
        compiler_params=_SC_PARAMS,
    )
    def k(qtab_h, ltab_h, qidx_h, lidx_h, qout_h, lout_h,
          qidx_v, lidx_v, qbuf, lbuf, sem_g, sem_o):
        w = _wid()
        pltpu.sync_copy(qidx_h.at[w], qidx_v)
        pltpu.sync_copy(lidx_h.at[w], lidx_v)
        qg = [None] * nchunks
        lg = [None] * nchunks
        qo = [None] * nchunks
        lo = [None] * nchunks

        def fire(tab_h, idx_v, buf, lst, j):
            lst[j] = pltpu.async_copy(
                tab_h.at[idx_v.at[j]],
                buf.at[pl.ds((j % NB) * CH, CH)], sem_g)

        for j in range(min(NB, nchunks)):
            fire(qtab_h, qidx_v, qbuf, qg, j)
            fire(ltab_h, lidx_v, lbuf, lg, j)
        for j in range(nchunks):
            qg[j].wait()
            qo[j] = pltpu.async_copy(
                qbuf.at[pl.ds((j % NB) * CH, CH)],
                qout_h.at[w, pl.ds(j * CH, CH)], sem_o)
            lg[j].wait()
            lo[j] = pltpu.async_copy(
                lbuf.at[pl.ds((j % NB) * CH, CH)],
                lout_h.at[w, pl.ds(j * CH, CH)], sem_o)
            if j + NB < nchunks:
                qo[j].wait()
                fire(qtab_h, qidx_v, qbuf, qg, j + NB)
                lo[j].wait()
                fire(ltab_h, lidx_v, lbuf, lg, j + NB)
        for j in range(max(0, nchunks - NB), nchunks):
            qo[j].wait()
            lo[j].wait()

    return k(qtab, ltab, qidx3, lidx3)


def _sc_gather_sum27(pss_flat, idx3):
    mesh = plsc.VectorSubcoreMesh(**_SC_MESH)

    @functools.partial(
        pl.kernel,
        out_type=jax.ShapeDtypeStruct((NW, QSUM_PER_W, D), jnp.float32),
        mesh=mesh,
        scratch_types=[
            pltpu.VMEM((QSUM_CH, CH), jnp.int32),
            pltpu.VMEM((QSUM_CH * CH, D), jnp.float32),
            pltpu.VMEM((QSUM_PER_W, D), jnp.float32),
            pltpu.SemaphoreType.DMA,
        ],
        compiler_params=_SC_PARAMS,
    )
    def k(pss_h, idx_h, out_h, idx_v, rows_v, out_v, sem):
        w = _wid()
        pltpu.sync_copy(idx_h.at[w], idx_v)
        cps = [
            pltpu.async_copy(pss_h.at[idx_v.at[j]],
                             rows_v.at[pl.ds(j * CH, CH)], sem)
            for j in range(QSUM_CH)
        ]
        for c in cps:
            c.wait()

        def qbody(q, _):
            base = q * PQ_
            acc0 = jnp.zeros((LANES,), jnp.float32)
            acc1 = jnp.zeros((LANES,), jnp.float32)
            for j in range(PQ_):
                acc0 = acc0 + rows_v[base + j, pl.ds(0, LANES)]
                acc1 = acc1 + rows_v[base + j, pl.ds(LANES, LANES)]
            out_v[q, pl.ds(0, LANES)] = acc0
            out_v[q, pl.ds(LANES, LANES)] = acc1
            return 0

        lax.fori_loop(0, QSUM_PER_W, qbody, 0)
        pltpu.sync_copy(out_v, out_h.at[w])

    return k(pss_flat, idx3)


def _sc_scalar_gather(table1d, idx2):
    nt = table1d.shape[0]
    npw = idx2.shape[1]
    mesh = plsc.VectorSubcoreMesh(**_SC_MESH)

    @functools.partial(
        pl.kernel,
        out_type=jax.ShapeDtypeStruct((NW, npw), jnp.float32),
        mesh=mesh,
        scratch_types=[
            pltpu.VMEM((nt,), jnp.float32),
            pltpu.VMEM((npw,), jnp.int32),
            pltpu.VMEM((npw,), jnp.float32),
        ],
        compiler_params=_SC_PARAMS,
    )
    def k(tab_h, idx_h, out_h, tab_v, idx_v, out_v):
        w = _wid()
        pltpu.sync_copy(tab_h, tab_v)
        pltpu.sync_copy(idx_h.at[w], idx_v)
        for g in range(npw // LANES):
            iv = idx_v[pl.ds(g * LANES, LANES)]
            out_v[pl.ds(g * LANES, LANES)] = plsc.load_gather(tab_v, [iv])
        pltpu.sync_copy(out_v, out_h.at[w])

    return k(table1d, idx2)


def _sc_load_sum(traffic1d, idx2):
    nt = traffic1d.shape[0]
    mesh = plsc.VectorSubcoreMesh(**_SC_MESH)

    @functools.partial(
        pl.kernel,
        out_type=jax.ShapeDtypeStruct((NW, 2 * LANES), jnp.float32),
        mesh=mesh,
        scratch_types=[
            pltpu.VMEM((nt,), jnp.float32),
            pltpu.VMEM((2 * PL_ * LANES,), jnp.int32),
            pltpu.VMEM((2 * LANES,), jnp.float32),
        ],
        compiler_params=_SC_PARAMS,
    )
    def k(tab_h, idx_h, out_h, tab_v, idx_v, out_v):
        w = _wid()
        pltpu.sync_copy(tab_h, tab_v)
        pltpu.sync_copy(idx_h.at[w], idx_v)
        for g in range(2):
            acc = jnp.zeros((LANES,), jnp.float32)
            for i in range(PL_):
                iv = idx_v[pl.ds((g * PL_ + i) * LANES, LANES)]
                acc = acc + plsc.load_gather(tab_v, [iv])
            out_v[pl.ds(g * LANES, LANES)] = acc
        pltpu.sync_copy(out_v, out_h.at[w])

    return k(traffic1d, idx2)


def _relu(x):
    return jnp.maximum(x, 0.0)


def _embed_body(path_in, sums, cap, pol_oh, queue_in,
                pw1, pb1, pw2, pb2, lw1, lb1, lw2, lb2, qw1, qb1, qw2, qb2,
                ps_o, ls_o, qs_o):
    x = path_in[...]
    h = _relu(x @ pw1[...] + pb1[...])
    ps_o[...] = _relu(h @ pw2[...] + pb2[...])
    load = sums[...] / cap[...]
    li = jnp.concatenate([load, pol_oh[...]], axis=1)
    h = _relu(li @ lw1[...] + lb1[...])
    ls_o[...] = _relu(h @ lw2[...] + lb2[...])
    qi = queue_in[...]
    h = _relu(qi @ qw1[...] + qb1[...])
    qs_o[...] = _relu(h @ qw2[...] + qb2[...])


def _tc_embed(path_in, sums, cap, pol_oh, queue_in, p):
    outs = [
        jax.ShapeDtypeStruct((P, D), jnp.float32),
        jax.ShapeDtypeStruct((L, D), jnp.float32),
        jax.ShapeDtypeStruct((QPAD, D), jnp.float32),
    ]
    return pl.pallas_call(_embed_body, out_shape=outs)(
        path_in, sums, cap, pol_oh, queue_in,
        p['pe_W1'], p['pe_b1'].reshape(1, D), p['pe_W2'], p['pe_b2'].reshape(1, D),
        p['le_W1'], p['le_b1'].reshape(1, D), p['le_W2'], p['le_b2'].reshape(1, D),
        p['qe_W1'], p['qe_b1'].reshape(1, D), p['qe_W2'], p['qe_b2'].reshape(1, D),
    )


def _gru_math(mx, mh, h):
    z = jax.nn.sigmoid(mx[:, 0:D] + mh[:, 0:D])
    r = jax.nn.sigmoid(mx[:, D:2 * D] + mh[:, D:2 * D])
    hh = jnp.tanh(mx[:, 2 * D:3 * D] + r * mh[:, 2 * D:3 * D])
    return z * h + (1.0 - z) * hh


def _path_gru_body(qg, lg, h0, Kq, Kl, R, b0, b1, pss_o, ht_o):
    h = h0[...]
    pss_o[0] = h
    for t in range(T):
        mx = qg[t] @ Kq[...] + lg[t] @ Kl[...] + b0[...]
        mh = h @ R[...] + b1[...]
        h = _gru_math(mx, mh, h)
        pss_o[t + 1] = h
    ht_o[...] = h


def _tc_path_gru(qgT, lgT, h0, Kq, Kl, R, b0, b1, bp=1000):
    ng = P // bp
    outs = [
        jax.ShapeDtypeStruct((T + 1, P, D), jnp.float32),
        jax.ShapeDtypeStruct((P, D), jnp.float32),
    ]
    return pl.pallas_call(
        _path_gru_body,
        grid=(ng,),
        in_specs=[
            pl.BlockSpec((T, bp, D), lambda i: (0, i, 0)),
            pl.BlockSpec((T, bp, D), lambda i: (0, i, 0)),
            pl.BlockSpec((bp, D), lambda i: (i, 0)),
            pl.BlockSpec((D, 3 * D), lambda i: (0, 0)),
            pl.BlockSpec((D, 3 * D), lambda i: (0, 0)),
            pl.BlockSpec((D, 3 * D), lambda i: (0, 0)),
            pl.BlockSpec((1, 3 * D), lambda i: (0, 0)),
            pl.BlockSpec((1, 3 * D), lambda i: (0, 0)),
        ],
        out_specs=[
            pl.BlockSpec((T + 1, bp, D), lambda i: (0, i, 0)),
            pl.BlockSpec((bp, D), lambda i: (i, 0)),
        ],
        out_shape=outs,
    )(qgT, lgT, h0, Kq, Kl, R, b0, b1)


def _queue_gru_body(xs, hs, K, R, b0, b1, out):
    mx = xs[...] @ K[...] + b0[...]
    mh = hs[...] @ R[...] + b1[...]
    out[...] = _gru_math(mx, mh, hs[...])


def _tc_queue_gru(xs, hs, K, R, b0, b1):
    return pl.pallas_call(
        _queue_gru_body,
        out_shape=jax.ShapeDtypeStruct((QPAD, D), jnp.float32),
    )(xs, hs, K, R, b0, b1)


def _link_gru_body(qg3, hs, K, R, b0, b1, out):
    h = hs[...]
    for j in range(QL_):
        x = qg3[pl.ds(j * 1024, L), :]
        mx = x @ K[...] + b0[...]
        mh = h @ R[...] + b1[...]
        h = _gru_math(mx, mh, h)
    out[...] = h


def _tc_link_gru(qg3_raw, hs, K, R, b0, b1):
    return pl.pallas_call(
        _link_gru_body,
        out_shape=jax.ShapeDtypeStruct((L, D), jnp.float32),
    )(qg3_raw, hs, K, R, b0, b1)


def _readout_body(pss, capg, lenr, tra, pkt, w1, b1, w2, b2, w3, b3, out):
    bp = out.shape[0]
    qd = jnp.zeros((bp, 1), jnp.float32)
    ts = jnp.zeros((bp, 1), jnp.float32)
    lenv = lenr[...]
    for t in range(T):
        x = pss[t + 1]
        h = _relu(x @ w1[...] + b1[...])
        h = _relu(h @ w2[...] + b2[...])
        occ = h @ w3[...] + b3[...]
        m = (lenv > t).astype(jnp.float32)
        c = capg[:, pl.ds(t, 1)]
        qd = qd + m * occ / c
        ts = ts + m / c
    out[...] = qd + (tra[...] / pkt[...]) * ts


def _tc_readout(pss, capg, length2, traffic, packets, p, bp=2000):
    ng = P // bp
    return pl.pallas_call(
        _readout_body,
        grid=(ng,),
        in_specs=[
            pl.BlockSpec((T + 1, bp, D), lambda i: (0, i, 0)),
            pl.BlockSpec((bp, T), lambda i: (i, 0)),
            pl.BlockSpec((bp, 1), lambda i: (i, 0)),
            pl.BlockSpec((bp, 1), lambda i: (i, 0)),
            pl.BlockSpec((bp, 1), lambda i: (i, 0)),
            pl.BlockSpec((D, 16), lambda i: (0, 0)),
            pl.BlockSpec((1, 16), lambda i: (0, 0)),
            pl.BlockSpec((16, 16), lambda i: (0, 0)),
            pl.BlockSpec((1, 16), lambda i: (0, 0)),
            pl.BlockSpec((16, 1), lambda i: (0, 0)),
            pl.BlockSpec((1, 1), lambda i: (0, 0)),
        ],
        out_specs=pl.BlockSpec((bp, 1), lambda i: (i, 0)),
        out_shape=jax.ShapeDtypeStruct((P, 1), jnp.float32),
    )(pss, capg, length2, traffic, packets,
      p['ro_W1'], p['ro_b1'].reshape(1, 16), p['ro_W2'], p['ro_b2'].reshape(1, 16),
      p['ro_W3'], p['ro_b3'].reshape(1, 1))


def _pad_to(x, n):
    return jnp.pad(x, ((0, n - x.shape[0]),))


def _chunk_idx(flat, total, nchunks):
    return _pad_to(flat, total).reshape(NW, nchunks, CH)


def kernel(traffic, packets, eq_lambda, avg_pkts_lambda, exp_max_factor,
           pkts_lambda_on, avg_t_off, avg_t_on, ar_a, sigma, capacity,
           queue_size, weight, length, model, policy, priority,
           queue_to_path, link_to_path, path_to_link, path_to_queue,
           queue_to_link, params):
    p = params

    nch = B_PT // (NW * CH)
    qtpT = jnp.pad(queue_to_path.T, ((0, 0), (0, PT_PAD - P)))
    ltpT = jnp.pad(link_to_path.T, ((0, 0), (0, PT_PAD - P)))
    qtp_idx3 = qtpT.reshape(NW, nch, CH)
    ltp_idx3 = ltpT.reshape(NW, nch, CH)

    ptq_flat = (path_to_queue[:, :, 1] * P + path_to_queue[:, :, 0])
    ptq_flat = jnp.pad(ptq_flat, ((0, QPAD - Q), (0, 0))).reshape(NW, QSUM_ROWS)
    ptq_idx3 = jnp.pad(ptq_flat, ((0, 0), (0, QSUM_CH * CH - QSUM_ROWS))
                       ).reshape(NW, QSUM_CH, CH)

    qtl_t = jnp.pad(queue_to_link.T, ((0, 0), (0, 1024 - L))).reshape(-1)
    qtl_idx3 = _pad_to(qtl_t, NW * CH).reshape(NW, 1, CH)

    capg_idx2 = _pad_to(link_to_path.reshape(-1), B_PT).reshape(NW, B_PT // NW)

    ptl0 = jnp.pad(path_to_link[:, :, 0], ((0, 1024 - L), (0, 0)))
    load_idx2 = ptl0.reshape(NW, 2, LANES, PL_).transpose(0, 1, 3, 2
                                                          ).reshape(NW, -1)

    model_oh = jax.nn.one_hot(model, MAX_MODELS, dtype=jnp.float32)
    policy_oh = jax.nn.one_hot(policy, NUM_POLICIES, dtype=jnp.float32)
    priority_oh = jax.nn.one_hot(priority, MAX_QUEUES, dtype=jnp.float32)
    path_in = jnp.concatenate([
        _zs(traffic, 'traffic'), _zs(packets, 'packets'), model_oh,
        _zs(eq_lambda, 'eq_lambda'), _zs(avg_pkts_lambda, 'avg_pkts_lambda'),
        _zs(exp_max_factor, 'exp_max_factor'), _zs(pkts_lambda_on, 'pkts_lambda_on'),
        _zs(avg_t_off, 'avg_t_off'), _zs(avg_t_on, 'avg_t_on'),
        _zs(ar_a, 'ar_a'), _zs(sigma, 'sigma')], axis=1)
    queue_in = jnp.concatenate([
        _zs(queue_size, 'queue_size'), priority_oh, weight], axis=1)
    queue_in = jnp.pad(queue_in, ((0, QPAD - Q), (0, 0)))

    sums_raw = _sc_load_sum(traffic.reshape(-1), load_idx2)
    sums = sums_raw.reshape(-1)[:L].reshape(L, 1)
    capg_raw = _sc_scalar_gather(capacity.reshape(-1), capg_idx2)
    capg = capg_raw.reshape(-1)[:P * T].reshape(P, T)

    path_state, link_state, queue_state = _tc_embed(
        path_in, sums, capacity, policy_oh, queue_in, p)

    pKq = p['path_K'][:D, :]
    pKl = p['path_K'][D:, :]
    pb0 = p['path_b'][0:1, :]
    pb1 = p['path_b'][1:2, :]
    qb0 = p['queue_b'][0:1, :]
    qb1 = p['queue_b'][1:2, :]
    lb0 = p['link_b'][0:1, :]
    lb1 = p['link_b'][1:2, :]

    pss = None
    for _ in range(ITERS):
        qg_raw, lg_raw = _sc_gather_rows2(
            queue_state, link_state, qtp_idx3, ltp_idx3)
        qgT = qg_raw.reshape(T, PT_PAD, D)
        lgT = lg_raw.reshape(T, PT_PAD, D)
        pss, path_state = _tc_path_gru(
            qgT, lgT, path_state, pKq, pKl, p['path_R'], pb0, pb1)
        psum_raw = _sc_gather_sum27(pss.reshape((T + 1) * P, D), ptq_idx3)
        queue_state = _tc_queue_gru(
            psum_raw.reshape(QPAD, D), queue_state,
            p['queue_K'], p['queue_R'], qb0, qb1)
        qg3l_raw = _sc_gather_rows(queue_state, qtl_idx3)
        link_state = _tc_link_gru(
            qg3l_raw.reshape(NW * CH, D), link_state,
            p['link_K'], p['link_R'], lb0, lb1)

    return _tc_readout(pss, capg, length.reshape(P, 1).astype(jnp.int32),
                       traffic, packets, p)


# speedup vs baseline: 2.5620x; 1.1000x over previous
import functools

import jax
import jax.numpy as jnp
from jax import lax
from jax.experimental import pallas as pl
from jax.experimental.pallas import tpu as pltpu
from jax.experimental.pallas import tpu_sc as plsc

P, L, Q, T = 10000, 1000, 3000, 8
PL_, PQ_, QL_ = 80, 27, 3
D = 32
MAX_MODELS, NUM_POLICIES, MAX_QUEUES = 7, 4, 3
ITERS = 8

NC, NS = 2, 16
NW = NC * NS
LANES = 16
CH = 128

QPAD = 3008
PT_PAD = 10240
B_PT = 81920
QSUM_PER_W = QPAD // NW
QSUM_ROWS = QSUM_PER_W * PQ_
QSUM_CH = 20

ZS = {'traffic': (1385.4058837890625, 859.8118896484375), 'packets': (1.4015231132507324, 0.8932565450668335), 'eq_lambda': (1350.97119140625, 858.316162109375), 'avg_pkts_lambda': (0.9117304086685181, 0.9723503589630127), 'exp_max_factor': (6.663637638092041, 4.715115070343018), 'pkts_lambda_on': (0.9116322994232178, 1.651275396347046), 'avg_t_off': (1.6649284362792969, 2.356407403945923), 'avg_t_on': (1.6649284362792969, 2.356407403945923), 'ar_a': (0.0, 1.0), 'sigma': (0.0, 1.0), 'capacity': (27611.091796875, 20090.62109375), 'queue_size': (30259.10546875, 21410.095703125)}


def _zs(x, name):
    m, s = ZS[name]
    return (x - m) / s


_SC_MESH = dict(core_axis_name="c", subcore_axis_name="s")
_SC_PARAMS = pltpu.CompilerParams(needs_layout_passes=False,
                                  use_tc_tiling_on_sc=False)


def _wid():
    return lax.axis_index("s") * NC + lax.axis_index("c")


def _sc_gather_rows(table, idx3):
    nchunks = idx3.shape[1]
    rows = nchunks * CH
    mesh = plsc.VectorSubcoreMesh(**_SC_MESH)

    @functools.partial(
        pl.kernel,
        out_type=jax.ShapeDtypeStruct((NW, rows, D), jnp.float32),
        mesh=mesh,
        scratch_types=[
            pltpu.VMEM((nchunks, CH), jnp.int32),
            pltpu.VMEM((rows, D), jnp.float32),
            pltpu.SemaphoreType.DMA,
            pltpu.SemaphoreType.DMA,
        ],
        compiler_params=_SC_PARAMS,
    )
    def k(table_h, idx_h, out_h, idx_v, rows_v, sem, sem_o):
        w = _wid()
        pltpu.sync_copy(idx_h.at[w], idx_v)
        cps = [
            pltpu.async_copy(table_h.at[idx_v.at[j]],
                             rows_v.at[pl.ds(j * CH, CH)], sem)
            for j in range(nchunks)
        ]
        outs = []
        for j in range(nchunks):
            cps[j].wait()
            outs.append(pltpu.async_copy(rows_v.at[pl.ds(j * CH, CH)],
                                         out_h.at[w, pl.ds(j * CH, CH)], sem_o))
        for c in outs:
            c.wait()

    return k(table, idx3)


def _sc_gather_rows2(qtab, ltab, qidx3, lidx3):
    nchunks = qidx3.shape[1]
    rows = nchunks * CH
    NB = 8
    mesh = plsc.VectorSubcoreMesh(**_SC_MESH)
    ot = jax.ShapeDtypeStruct((NW, rows, D), jnp.bfloat16)

    @functools.partial(
        pl.kernel,
        out_type=(ot, ot),
        mesh=mesh,
        scratch_types=[
            pltpu.VMEM((nchunks, CH), jnp.int32),
            pltpu.VMEM((nchunks, CH), jnp.int32),
            pltpu.VMEM((NB * CH, D), jnp.bfloat16),
            pltpu.VMEM((NB * CH, D), jnp.bfloat16),
            pltpu.SemaphoreType.DMA,
            pltpu.SemaphoreType.DMA,
        ],
        compiler_params=_SC_PARAMS,
    )
    def k(qtab_h, ltab_h, qidx_h, lidx_h, qout_h, lout_h,
          qidx_v, lidx_v, qbuf, lbuf, sem_g, sem_o):
        w = _wid()
        pltpu.sync_copy(qidx_h.at[w], qidx_v)
        pltpu.sync_copy(lidx_h.at[w], lidx_v)
        qg = [None] * nchunks
        lg = [None] * nchunks
        qo = [None] * nchunks
        lo = [None] * nchunks

        def fire(tab_h, idx_v, buf, lst, j):
            lst[j] = pltpu.async_copy(
                tab_h.at[idx_v.at[j]],
                buf.at[pl.ds((j % NB) * CH, CH)], sem_g)

        for j in range(min(NB, nchunks)):
            fire(qtab_h, qidx_v, qbuf, qg, j)
            fire(ltab_h, lidx_v, lbuf, lg, j)
        for j in range(nchunks):
            qg[j].wait()
            qo[j] = pltpu.async_copy(
                qbuf.at[pl.ds((j % NB) * CH, CH)],
                qout_h.at[w, pl.ds(j * CH, CH)], sem_o)
            lg[j].wait()
            lo[j] = pltpu.async_copy(
                lbuf.at[pl.ds((j % NB) * CH, CH)],
                lout_h.at[w, pl.ds(j * CH, CH)], sem_o)
            if j + NB < nchunks:
                qo[j].wait()
                fire(qtab_h, qidx_v, qbuf, qg, j + NB)
                lo[j].wait()
                fire(ltab_h, lidx_v, lbuf, lg, j + NB)
        for j in range(max(0, nchunks - NB), nchunks):
            qo[j].wait()
            lo[j].wait()

    return k(qtab, ltab, qidx3, lidx3)


def _sc_gather_sum27(pss_flat, idx3):
    mesh = plsc.VectorSubcoreMesh(**_SC_MESH)

    @functools.partial(
        pl.kernel,
        out_type=jax.ShapeDtypeStruct((NW, QSUM_PER_W, D), jnp.float32),
        mesh=mesh,
        scratch_types=[
            pltpu.VMEM((QSUM_CH, CH), jnp.int32),
            pltpu.VMEM((QSUM_CH * CH, D), jnp.float32),
            pltpu.VMEM((QSUM_PER_W, D), jnp.float32),
            pltpu.SemaphoreType.DMA,
        ],
        compiler_params=_SC_PARAMS,
    )
    def k(pss_h, idx_h, out_h, idx_v, rows_v, out_v, sem):
        w = _wid()
        pltpu.sync_copy(idx_h.at[w], idx_v)
        cps = [
            pltpu.async_copy(pss_h.at[idx_v.at[j]],
                             rows_v.at[pl.ds(j * CH, CH)], sem)
            for j in range(QSUM_CH)
        ]
        for c in cps:
            c.wait()

        def qbody(q, _):
            base = q * PQ_
            acc0 = jnp.zeros((LANES,), jnp.float32)
            acc1 = jnp.zeros((LANES,), jnp.float32)
            for j in range(PQ_):
                acc0 = acc0 + rows_v[base + j, pl.ds(0, LANES)]
                acc1 = acc1 + rows_v[base + j, pl.ds(LANES, LANES)]
            out_v[q, pl.ds(0, LANES)] = acc0
            out_v[q, pl.ds(LANES, LANES)] = acc1
            return 0

        lax.fori_loop(0, QSUM_PER_W, qbody, 0)
        pltpu.sync_copy(out_v, out_h.at[w])

    return k(pss_flat, idx3)


def _sc_scalar_gather(table1d, idx2):
    nt = table1d.shape[0]
    npw = idx2.shape[1]
    mesh = plsc.VectorSubcoreMesh(**_SC_MESH)

    @functools.partial(
        pl.kernel,
        out_type=jax.ShapeDtypeStruct((NW, npw), jnp.float32),
        mesh=mesh,
        scratch_types=[
            pltpu.VMEM((nt,), jnp.float32),
            pltpu.VMEM((npw,), jnp.int32),
            pltpu.VMEM((npw,), jnp.float32),
        ],
        compiler_params=_SC_PARAMS,
    )
    def k(tab_h, idx_h, out_h, tab_v, idx_v, out_v):
        w = _wid()
        pltpu.sync_copy(tab_h, tab_v)
        pltpu.sync_copy(idx_h.at[w], idx_v)
        for g in range(npw // LANES):
            iv = idx_v[pl.ds(g * LANES, LANES)]
            out_v[pl.ds(g * LANES, LANES)] = plsc.load_gather(tab_v, [iv])
        pltpu.sync_copy(out_v, out_h.at[w])

    return k(table1d, idx2)


def _sc_load_sum(traffic1d, idx2):
    nt = traffic1d.shape[0]
    mesh = plsc.VectorSubcoreMesh(**_SC_MESH)

    @functools.partial(
        pl.kernel,
        out_type=jax.ShapeDtypeStruct((NW, 2 * LANES), jnp.float32),
        mesh=mesh,
        scratch_types=[
            pltpu.VMEM((nt,), jnp.float32),
            pltpu.VMEM((2 * PL_ * LANES,), jnp.int32),
            pltpu.VMEM((2 * LANES,), jnp.float32),
        ],
        compiler_params=_SC_PARAMS,
    )
    def k(tab_h, idx_h, out_h, tab_v, idx_v, out_v):
        w = _wid()
        pltpu.sync_copy(tab_h, tab_v)
        pltpu.sync_copy(idx_h.at[w], idx_v)
        for g in range(2):
            acc = jnp.zeros((LANES,), jnp.float32)
            for i in range(PL_):
                iv = idx_v[pl.ds((g * PL_ + i) * LANES, LANES)]
                acc = acc + plsc.load_gather(tab_v, [iv])
            out_v[pl.ds(g * LANES, LANES)] = acc
        pltpu.sync_copy(out_v, out_h.at[w])

    return k(traffic1d, idx2)


def _relu(x):
    return jnp.maximum(x, 0.0)


def _embed_body(path_in, sums, cap, pol_oh, queue_in,
                pw1, pb1, pw2, pb2, lw1, lb1, lw2, lb2, qw1, qb1, qw2, qb2,
                ps_o, ls_o, qs_o, lsb_o, qsb_o):
    x = path_in[...]
    h = _relu(x @ pw1[...] + pb1[...])
    ps_o[...] = _relu(h @ pw2[...] + pb2[...])
    load = sums[...] / cap[...]
    li = jnp.concatenate([load, pol_oh[...]], axis=1)
    h = _relu(li @ lw1[...] + lb1[...])
    ls = _relu(h @ lw2[...] + lb2[...])
    ls_o[...] = ls
    lsb_o[...] = ls.astype(jnp.bfloat16)
    qi = queue_in[...]
    h = _relu(qi @ qw1[...] + qb1[...])
    qs = _relu(h @ qw2[...] + qb2[...])
    qs_o[...] = qs
    qsb_o[...] = qs.astype(jnp.bfloat16)


def _tc_embed(path_in, sums, cap, pol_oh, queue_in, p):
    outs = [
        jax.ShapeDtypeStruct((P, D), jnp.float32),
        jax.ShapeDtypeStruct((L, D), jnp.float32),
        jax.ShapeDtypeStruct((QPAD, D), jnp.float32),
        jax.ShapeDtypeStruct((L, D), jnp.bfloat16),
        jax.ShapeDtypeStruct((QPAD, D), jnp.bfloat16),
    ]
    return pl.pallas_call(_embed_body, out_shape=outs)(
        path_in, sums, cap, pol_oh, queue_in,
        p['pe_W1'], p['pe_b1'].reshape(1, D), p['pe_W2'], p['pe_b2'].reshape(1, D),
        p['le_W1'], p['le_b1'].reshape(1, D), p['le_W2'], p['le_b2'].reshape(1, D),
        p['qe_W1'], p['qe_b1'].reshape(1, D), p['qe_W2'], p['qe_b2'].reshape(1, D),
    )


def _gru_math(mx, mh, h):
    z = jax.nn.sigmoid(mx[:, 0:D] + mh[:, 0:D])
    r = jax.nn.sigmoid(mx[:, D:2 * D] + mh[:, D:2 * D])
    hh = jnp.tanh(mx[:, 2 * D:3 * D] + r * mh[:, 2 * D:3 * D])
    return z * h + (1.0 - z) * hh


def _path_gru_body(qg, lg, h0, Kq, Kl, R, b0, b1, pss_o, ht_o):
    h = h0[...]
    pss_o[0] = h
    for t in range(T):
        qt = qg[t].astype(jnp.float32)
        lt = lg[t].astype(jnp.float32)
        mx = qt @ Kq[...] + lt @ Kl[...] + b0[...]
        mh = h @ R[...] + b1[...]
        h = _gru_math(mx, mh, h)
        pss_o[t + 1] = h
    ht_o[...] = h


def _tc_path_gru(qgT, lgT, h0, Kq, Kl, R, b0, b1, bp=1000):
    ng = P // bp
    outs = [
        jax.ShapeDtypeStruct((T + 1, P, D), jnp.float32),
        jax.ShapeDtypeStruct((P, D), jnp.float32),
    ]
    return pl.pallas_call(
        _path_gru_body,
        grid=(ng,),
        in_specs=[
            pl.BlockSpec((T, bp, D), lambda i: (0, i, 0)),
            pl.BlockSpec((T, bp, D), lambda i: (0, i, 0)),
            pl.BlockSpec((bp, D), lambda i: (i, 0)),
            pl.BlockSpec((D, 3 * D), lambda i: (0, 0)),
            pl.BlockSpec((D, 3 * D), lambda i: (0, 0)),
            pl.BlockSpec((D, 3 * D), lambda i: (0, 0)),
            pl.BlockSpec((1, 3 * D), lambda i: (0, 0)),
            pl.BlockSpec((1, 3 * D), lambda i: (0, 0)),
        ],
        out_specs=[
            pl.BlockSpec((T + 1, bp, D), lambda i: (0, i, 0)),
            pl.BlockSpec((bp, D), lambda i: (i, 0)),
        ],
        out_shape=outs,
    )(qgT, lgT, h0, Kq, Kl, R, b0, b1)


def _queue_gru_body(xs, hs, K, R, b0, b1, out, outb):
    mx = xs[...] @ K[...] + b0[...]
    mh = hs[...] @ R[...] + b1[...]
    h = _gru_math(mx, mh, hs[...])
    out[...] = h
    outb[...] = h.astype(jnp.bfloat16)


def _tc_queue_gru(xs, hs, K, R, b0, b1):
    return pl.pallas_call(
        _queue_gru_body,
        out_shape=[jax.ShapeDtypeStruct((QPAD, D), jnp.float32),
                   jax.ShapeDtypeStruct((QPAD, D), jnp.bfloat16)],
    )(xs, hs, K, R, b0, b1)


def _link_gru_body(qg3, hs, K, R, b0, b1, out, outb):
    h = hs[...]
    for j in range(QL_):
        x = qg3[pl.ds(j * 1024, L), :]
        mx = x @ K[...] + b0[...]
        mh = h @ R[...] + b1[...]
        h = _gru_math(mx, mh, h)
    out[...] = h
    outb[...] = h.astype(jnp.bfloat16)


def _tc_link_gru(qg3_raw, hs, K, R, b0, b1):
    return pl.pallas_call(
        _link_gru_body,
        out_shape=[jax.ShapeDtypeStruct((L, D), jnp.float32),
                   jax.ShapeDtypeStruct((L, D), jnp.bfloat16)],
    )(qg3_raw, hs, K, R, b0, b1)


def _readout_body(pss, capg, lenr, tra, pkt, w1, b1, w2, b2, w3, b3, out):
    bp = out.shape[0]
    qd = jnp.zeros((bp, 1), jnp.float32)
    ts = jnp.zeros((bp, 1), jnp.float32)
    lenv = lenr[...]
    for t in range(T):
        x = pss[t + 1]
        h = _relu(x @ w1[...] + b1[...])
        h = _relu(h @ w2[...] + b2[...])
        occ = h @ w3[...] + b3[...]
        m = (lenv > t).astype(jnp.float32)
        c = capg[:, pl.ds(t, 1)]
        qd = qd + m * occ / c
        ts = ts + m / c
    out[...] = qd + (tra[...] / pkt[...]) * ts


def _tc_readout(pss, capg, length2, traffic, packets, p, bp=2000):
    ng = P // bp
    return pl.pallas_call(
        _readout_body,
        grid=(ng,),
        in_specs=[
            pl.BlockSpec((T + 1, bp, D), lambda i: (0, i, 0)),
            pl.BlockSpec((bp, T), lambda i: (i, 0)),
            pl.BlockSpec((bp, 1), lambda i: (i, 0)),
            pl.BlockSpec((bp, 1), lambda i: (i, 0)),
            pl.BlockSpec((bp, 1), lambda i: (i, 0)),
            pl.BlockSpec((D, 16), lambda i: (0, 0)),
            pl.BlockSpec((1, 16), lambda i: (0, 0)),
            pl.BlockSpec((16, 16), lambda i: (0, 0)),
            pl.BlockSpec((1, 16), lambda i: (0, 0)),
            pl.BlockSpec((16, 1), lambda i: (0, 0)),
            pl.BlockSpec((1, 1), lambda i: (0, 0)),
        ],
        out_specs=pl.BlockSpec((bp, 1), lambda i: (i, 0)),
        out_shape=jax.ShapeDtypeStruct((P, 1), jnp.float32),
    )(pss, capg, length2, traffic, packets,
      p['ro_W1'], p['ro_b1'].reshape(1, 16), p['ro_W2'], p['ro_b2'].reshape(1, 16),
      p['ro_W3'], p['ro_b3'].reshape(1, 1))


def _pad_to(x, n):
    return jnp.pad(x, ((0, n - x.shape[0]),))


def _chunk_idx(flat, total, nchunks):
    return _pad_to(flat, total).reshape(NW, nchunks, CH)


def kernel(traffic, packets, eq_lambda, avg_pkts_lambda, exp_max_factor,
           pkts_lambda_on, avg_t_off, avg_t_on, ar_a, sigma, capacity,
           queue_size, weight, length, model, policy, priority,
           queue_to_path, link_to_path, path_to_link, path_to_queue,
           queue_to_link, params):
    p = params

    nch = B_PT // (NW * CH)
    qtpT = jnp.pad(queue_to_path.T, ((0, 0), (0, PT_PAD - P)))
    ltpT = jnp.pad(link_to_path.T, ((0, 0), (0, PT_PAD - P)))
    qtp_idx3 = qtpT.reshape(NW, nch, CH)
    ltp_idx3 = ltpT.reshape(NW, nch, CH)

    ptq_flat = (path_to_queue[:, :, 1] * P + path_to_queue[:, :, 0])
    ptq_flat = jnp.pad(ptq_flat, ((0, QPAD - Q), (0, 0))).reshape(NW, QSUM_ROWS)
    ptq_idx3 = jnp.pad(ptq_flat, ((0, 0), (0, QSUM_CH * CH - QSUM_ROWS))
                       ).reshape(NW, QSUM_CH, CH)

    qtl_t = jnp.pad(queue_to_link.T, ((0, 0), (0, 1024 - L))).reshape(-1)
    qtl_idx3 = _pad_to(qtl_t, NW * CH).reshape(NW, 1, CH)

    capg_idx2 = _pad_to(link_to_path.reshape(-1), B_PT).reshape(NW, B_PT // NW)

    ptl0 = jnp.pad(path_to_link[:, :, 0], ((0, 1024 - L), (0, 0)))
    load_idx2 = ptl0.reshape(NW, 2, LANES, PL_).transpose(0, 1, 3, 2
                                                          ).reshape(NW, -1)

    model_oh = jax.nn.one_hot(model, MAX_MODELS, dtype=jnp.float32)
    policy_oh = jax.nn.one_hot(policy, NUM_POLICIES, dtype=jnp.float32)
    priority_oh = jax.nn.one_hot(priority, MAX_QUEUES, dtype=jnp.float32)
    path_in = jnp.concatenate([
        _zs(traffic, 'traffic'), _zs(packets, 'packets'), model_oh,
        _zs(eq_lambda, 'eq_lambda'), _zs(avg_pkts_lambda, 'avg_pkts_lambda'),
        _zs(exp_max_factor, 'exp_max_factor'), _zs(pkts_lambda_on, 'pkts_lambda_on'),
        _zs(avg_t_off, 'avg_t_off'), _zs(avg_t_on, 'avg_t_on'),
        _zs(ar_a, 'ar_a'), _zs(sigma, 'sigma')], axis=1)
    queue_in = jnp.concatenate([
        _zs(queue_size, 'queue_size'), priority_oh, weight], axis=1)
    queue_in = jnp.pad(queue_in, ((0, QPAD - Q), (0, 0)))

    sums_raw = _sc_load_sum(traffic.reshape(-1), load_idx2)
    sums = sums_raw.reshape(-1)[:L].reshape(L, 1)
    capg_raw = _sc_scalar_gather(capacity.reshape(-1), capg_idx2)
    capg = capg_raw.reshape(-1)[:P * T].reshape(P, T)

    path_state, link_state, queue_state, link_b16, queue_b16 = _tc_embed(
        path_in, sums, capacity, policy_oh, queue_in, p)

    pKq = p['path_K'][:D, :]
    pKl = p['path_K'][D:, :]
    pb0 = p['path_b'][0:1, :]
    pb1 = p['path_b'][1:2, :]
    qb0 = p['queue_b'][0:1, :]
    qb1 = p['queue_b'][1:2, :]
    lb0 = p['link_b'][0:1, :]
    lb1 = p['link_b'][1:2, :]

    pss = None
    for _ in range(ITERS):
        qg_raw, lg_raw = _sc_gather_rows2(
            queue_b16, link_b16, qtp_idx3, ltp_idx3)
        qgT = qg_raw.reshape(T, PT_PAD, D)
        lgT = lg_raw.reshape(T, PT_PAD, D)
        pss, path_state = _tc_path_gru(
            qgT, lgT, path_state, pKq, pKl, p['path_R'], pb0, pb1)
        psum_raw = _sc_gather_sum27(pss.reshape((T + 1) * P, D), ptq_idx3)
        queue_state, queue_b16 = _tc_queue_gru(
            psum_raw.reshape(QPAD, D), queue_state,
            p['queue_K'], p['queue_R'], qb0, qb1)
        qg3l_raw = _sc_gather_rows(queue_state, qtl_idx3)
        link_state, link_b16 = _tc_link_gru(
            qg3l_raw.reshape(NW * CH, D), link_state,
            p['link_K'], p['link_R'], lb0, lb1)

    return _tc_readout(pss, capg, length.reshape(P, 1).astype(jnp.int32),
                       traffic, packets, p)
